# baseline (speedup 1.0000x reference)
import functools
import jax
import jax.numpy as jnp
from jax import lax
from jax.experimental import pallas as pl
from jax.experimental.pallas import tpu as pltpu
from jax.experimental.pallas import tpu_sc as plsc

TILE = 16
SEQ = 65536
DIM = 128
HALF = DIM // 2
NTAB = 512
NC, NS, L = 2, 16, 16
NW = NC * NS
ROWS = SEQ // NW
SCAN = SEQ // NS
VPS = SCAN // L
VPW = ROWS // L
BLK = 128
NBLK = ROWS // BLK
NBUF = 4
TROWS = NTAB // NS

_mesh = plsc.VectorSubcoreMesh(
    core_axis_name="c", subcore_axis_name="s", num_cores=NC, num_subcores=NS
)


def _lane_shuffle(v, idx):
    return lax.gather(
        v,
        idx[:, None],
        dimension_numbers=lax.GatherDimensionNumbers(
            offset_dims=(), collapsed_slice_dims=(0,), start_index_map=(0,)
        ),
        slice_sizes=(1,),
        mode=lax.GatherScatterMode.PROMISE_IN_BOUNDS,
    )


@functools.partial(
    pl.kernel,
    out_type=jax.ShapeDtypeStruct((SEQ, DIM), jnp.float32),
    mesh=_mesh,
    scratch_types=[
        pltpu.VMEM((SCAN,), jnp.int32),
        pltpu.VMEM((SCAN,), jnp.int32),
        pltpu.VMEM((ROWS,), jnp.int32),
        pltpu.VMEM((ROWS,), jnp.int32),
        pltpu.VMEM((2 * NS * L,), jnp.int32),
        pltpu.VMEM((2 * L,), jnp.int32),
        pltpu.VMEM((NBUF, BLK, DIM), jnp.float32),
        pltpu.VMEM_SHARED((NTAB, DIM), jnp.float32),
        pltpu.VMEM_SHARED((NTAB, DIM), jnp.float32),
        pltpu.VMEM_SHARED((2 * NS * L,), jnp.int32),
        pltpu.SemaphoreType.DMA((NBUF,)),
        pltpu.SemaphoreType.DMA((NBUF,)),
        pltpu.SemaphoreType.DMA((NBUF,)),
    ],
    compiler_params=pltpu.CompilerParams(use_tc_tiling_on_sc=True),
)
def _emb_kernel(x_hbm, c1_hbm, c2_hbm, taba_hbm, tabb_hbm, out_hbm,
                cbuf1, cbuf2, idx1buf, idx2buf, mbuf, mv, xbuf,
                taba_sh, tabb_sh, min_sh, lsem, gsem, ssem):
    cid = lax.axis_index("c")
    sid = lax.axis_index("s")
    wid = sid * NC + cid
    rbase = wid * ROWS

    for jw in range(2):
        pltpu.async_copy(
            x_hbm.at[pl.ds(rbase + jw * BLK, BLK)], xbuf.at[jw],
            lsem.at[jw],
        )

    pltpu.sync_copy(taba_hbm.at[pl.ds(sid * TROWS, TROWS)],
                    taba_sh.at[pl.ds(sid * TROWS, TROWS)])
    pltpu.sync_copy(tabb_hbm.at[pl.ds(sid * TROWS, TROWS)],
                    tabb_sh.at[pl.ds(sid * TROWS, TROWS)])

    pltpu.sync_copy(c1_hbm.at[pl.ds(sid * SCAN, SCAN)], cbuf1)
    pltpu.sync_copy(c2_hbm.at[pl.ds(sid * SCAN, SCAN)], cbuf2)

    def body(i, ms):
        m1, m2 = ms
        return (jnp.minimum(m1, cbuf1[pl.ds(i * L, L)]),
                jnp.minimum(m2, cbuf2[pl.ds(i * L, L)]))

    m1, m2 = lax.fori_loop(1, VPS, body,
                           (cbuf1[pl.ds(0, L)], cbuf2[pl.ds(0, L)]))
    mv[pl.ds(0, L)] = m1
    mv[pl.ds(L, L)] = m2
    pltpu.sync_copy(mv, min_sh.at[pl.ds(sid * 2 * L, 2 * L)])
    plsc.subcore_barrier()

    pltpu.sync_copy(min_sh, mbuf)

    def mbody(i, ms):
        m1, m2 = ms
        return (jnp.minimum(m1, mbuf[pl.ds(i * 2 * L, L)]),
                jnp.minimum(m2, mbuf[pl.ds(i * 2 * L + L, L)]))

    m1, m2 = lax.fori_loop(1, NS, mbody,
                           (mbuf[pl.ds(0, L)], mbuf[pl.ds(L, L)]))
    iota = lax.iota(jnp.int32, L)
    for d in (1, 2, 4, 8):
        perm = jnp.bitwise_xor(iota, d)
        m1 = jnp.minimum(m1, _lane_shuffle(m1, perm))
        m2 = jnp.minimum(m2, _lane_shuffle(m2, perm))

    cb = cid * ROWS

    @pl.loop(0, VPW)
    def _(i):
        idx1buf[pl.ds(i * L, L)] = lax.shift_right_arithmetic(
            cbuf1[pl.ds(cb + i * L, L)] - m1, 4)
        idx2buf[pl.ds(i * L, L)] = lax.shift_right_arithmetic(
            cbuf2[pl.ds(cb + i * L, L)] - m2, 4)

    @pl.loop(0, NBLK + 2)
    def _(j):
        @pl.when(j >= 2)
        def _():
            jj = j - 2
            b = jj & (NBUF - 1)
            pltpu.make_async_copy(
                x_hbm.at[pl.ds(rbase + jj * BLK, BLK)], xbuf.at[b],
                gsem.at[b],
            ).wait()
            pltpu.make_async_copy(
                x_hbm.at[pl.ds(rbase + jj * BLK, BLK)], xbuf.at[b],
                gsem.at[b],
            ).wait()
            pltpu.async_copy(
                xbuf.at[b], out_hbm.at[pl.ds(rbase + jj * BLK, BLK)],
                ssem.at[b],
            )

        @pl.when((j >= 1) & (j <= NBLK))
        def _():
            jj = j - 1
            b = jj & (NBUF - 1)
            pltpu.make_async_copy(
                x_hbm.at[pl.ds(rbase + jj * BLK, BLK)], xbuf.at[b],
                lsem.at[b],
            ).wait()
            pltpu.async_copy(
                taba_sh.at[idx1buf.at[pl.ds(jj * BLK, BLK)]], xbuf.at[b],
                gsem.at[b], add=True,
            )
            pltpu.async_copy(
                tabb_sh.at[idx2buf.at[pl.ds(jj * BLK, BLK)]], xbuf.at[b],
                gsem.at[b], add=True,
            )

        @pl.when((j >= 2) & (j < NBLK))
        def _():
            b = j & (NBUF - 1)

            @pl.when(j >= NBUF)
            def _():
                pltpu.make_async_copy(
                    xbuf.at[b],
                    out_hbm.at[pl.ds(rbase + (j - NBUF) * BLK, BLK)],
                    ssem.at[b],
                ).wait()

            pltpu.async_copy(
                x_hbm.at[pl.ds(rbase + j * BLK, BLK)], xbuf.at[b],
                lsem.at[b],
            )

    @pl.loop(NBLK, NBLK + NBUF)
    def _(j):
        b = j & (NBUF - 1)
        pltpu.make_async_copy(
            xbuf.at[b], out_hbm.at[pl.ds(rbase + (j - NBUF) * BLK, BLK)],
            ssem.at[b],
        ).wait()


def kernel(x, coords, emb1, emb2):
    c1 = coords[:, 0]
    c2 = coords[:, 1]
    zeros = jnp.zeros((NTAB, HALF), jnp.float32)
    taba = jnp.concatenate([emb1, zeros], axis=1)
    tabb = jnp.concatenate([zeros, emb2], axis=1)
    return _emb_kernel(x, c1, c2, taba, tabb)


# speedup vs baseline: 1.0082x; 1.0082x over previous
import functools
import jax
import jax.numpy as jnp
from jax import lax
from jax.experimental import pallas as pl
from jax.experimental.pallas import tpu as pltpu
from jax.experimental.pallas import tpu_sc as plsc

TILE = 16
SEQ = 65536
DIM = 128
HALF = DIM // 2
NTAB = 512
NC, NS, L = 2, 16, 16
NW = NC * NS
ROWS = SEQ // NW
SCAN = SEQ // NS
VPS = SCAN // L
VPW = ROWS // L
BLK = 128
NBLK = ROWS // BLK
NBUF = 4
TROWS = NTAB // NS

_mesh = plsc.VectorSubcoreMesh(
    core_axis_name="c", subcore_axis_name="s", num_cores=NC, num_subcores=NS
)


def _lane_shuffle(v, idx):
    return lax.gather(
        v,
        idx[:, None],
        dimension_numbers=lax.GatherDimensionNumbers(
            offset_dims=(), collapsed_slice_dims=(0,), start_index_map=(0,)
        ),
        slice_sizes=(1,),
        mode=lax.GatherScatterMode.PROMISE_IN_BOUNDS,
    )


@functools.partial(
    pl.kernel,
    out_type=jax.ShapeDtypeStruct((SEQ, DIM), jnp.float32),
    mesh=_mesh,
    scratch_types=[
        pltpu.VMEM((SCAN,), jnp.int32),
        pltpu.VMEM((SCAN,), jnp.int32),
        pltpu.VMEM((ROWS,), jnp.int32),
        pltpu.VMEM((ROWS,), jnp.int32),
        pltpu.VMEM((2 * NS * L,), jnp.int32),
        pltpu.VMEM((2 * L,), jnp.int32),
        pltpu.VMEM((NBUF, BLK, DIM), jnp.float32),
        pltpu.VMEM_SHARED((NTAB, DIM), jnp.float32),
        pltpu.VMEM_SHARED((NTAB, DIM), jnp.float32),
        pltpu.VMEM_SHARED((2 * NS * L,), jnp.int32),
        pltpu.SemaphoreType.DMA((NBUF,)),
        pltpu.SemaphoreType.DMA((NBUF,)),
        pltpu.SemaphoreType.DMA((NBUF,)),
    ],
    compiler_params=pltpu.CompilerParams(use_tc_tiling_on_sc=True),
)
def _emb_kernel(x_hbm, c1_hbm, c2_hbm, taba_hbm, tabb_hbm, out_hbm,
                cbuf1, cbuf2, idx1buf, idx2buf, mbuf, mv, xbuf,
                taba_sh, tabb_sh, min_sh, lsem, gsem, ssem):
    cid = lax.axis_index("c")
    sid = lax.axis_index("s")
    wid = sid * NC + cid
    rbase = wid * ROWS

    for jw in range(2):
        pltpu.async_copy(
            x_hbm.at[pl.ds(rbase + jw * BLK, BLK)], xbuf.at[jw],
            lsem.at[jw],
        )

    pltpu.sync_copy(taba_hbm.at[pl.ds(sid * TROWS, TROWS)],
                    taba_sh.at[pl.ds(sid * TROWS, TROWS)])
    pltpu.sync_copy(tabb_hbm.at[pl.ds(sid * TROWS, TROWS)],
                    tabb_sh.at[pl.ds(sid * TROWS, TROWS)])

    pltpu.sync_copy(c1_hbm.at[pl.ds(sid * SCAN, SCAN)], cbuf1)
    pltpu.sync_copy(c2_hbm.at[pl.ds(sid * SCAN, SCAN)], cbuf2)

    def body(i, ms):
        m1, m2 = ms
        return (jnp.minimum(m1, cbuf1[pl.ds(i * L, L)]),
                jnp.minimum(m2, cbuf2[pl.ds(i * L, L)]))

    m1, m2 = lax.fori_loop(1, VPS, body,
                           (cbuf1[pl.ds(0, L)], cbuf2[pl.ds(0, L)]),
                           unroll=8)
    mv[pl.ds(0, L)] = m1
    mv[pl.ds(L, L)] = m2
    pltpu.sync_copy(mv, min_sh.at[pl.ds(sid * 2 * L, 2 * L)])
    plsc.subcore_barrier()

    pltpu.sync_copy(min_sh, mbuf)

    def mbody(i, ms):
        m1, m2 = ms
        return (jnp.minimum(m1, mbuf[pl.ds(i * 2 * L, L)]),
                jnp.minimum(m2, mbuf[pl.ds(i * 2 * L + L, L)]))

    m1, m2 = lax.fori_loop(1, NS, mbody,
                           (mbuf[pl.ds(0, L)], mbuf[pl.ds(L, L)]))
    iota = lax.iota(jnp.int32, L)
    for d in (1, 2, 4, 8):
        perm = jnp.bitwise_xor(iota, d)
        m1 = jnp.minimum(m1, _lane_shuffle(m1, perm))
        m2 = jnp.minimum(m2, _lane_shuffle(m2, perm))

    cb = cid * ROWS

    @pl.loop(0, VPW, unroll=8)
    def _(i):
        idx1buf[pl.ds(i * L, L)] = lax.shift_right_arithmetic(
            cbuf1[pl.ds(cb + i * L, L)] - m1, 4)
        idx2buf[pl.ds(i * L, L)] = lax.shift_right_arithmetic(
            cbuf2[pl.ds(cb + i * L, L)] - m2, 4)

    @pl.loop(0, NBLK + 2)
    def _(j):
        @pl.when(j >= 2)
        def _():
            jj = j - 2
            b = jj & (NBUF - 1)
            pltpu.make_async_copy(
                x_hbm.at[pl.ds(rbase + jj * BLK, BLK)], xbuf.at[b],
                gsem.at[b],
            ).wait()
            pltpu.make_async_copy(
                x_hbm.at[pl.ds(rbase + jj * BLK, BLK)], xbuf.at[b],
                gsem.at[b],
            ).wait()
            pltpu.async_copy(
                xbuf.at[b], out_hbm.at[pl.ds(rbase + jj * BLK, BLK)],
                ssem.at[b],
            )

        @pl.when((j >= 1) & (j <= NBLK))
        def _():
            jj = j - 1
            b = jj & (NBUF - 1)
            pltpu.make_async_copy(
                x_hbm.at[pl.ds(rbase + jj * BLK, BLK)], xbuf.at[b],
                lsem.at[b],
            ).wait()
            pltpu.async_copy(
                taba_sh.at[idx1buf.at[pl.ds(jj * BLK, BLK)]], xbuf.at[b],
                gsem.at[b], add=True,
            )
            pltpu.async_copy(
                tabb_sh.at[idx2buf.at[pl.ds(jj * BLK, BLK)]], xbuf.at[b],
                gsem.at[b], add=True,
            )

        @pl.when((j >= 2) & (j < NBLK))
        def _():
            b = j & (NBUF - 1)

            @pl.when(j >= NBUF)
            def _():
                pltpu.make_async_copy(
                    xbuf.at[b],
                    out_hbm.at[pl.ds(rbase + (j - NBUF) * BLK, BLK)],
                    ssem.at[b],
                ).wait()

            pltpu.async_copy(
                x_hbm.at[pl.ds(rbase + j * BLK, BLK)], xbuf.at[b],
                lsem.at[b],
            )

    @pl.loop(NBLK, NBLK + NBUF)
    def _(j):
        b = j & (NBUF - 1)
        pltpu.make_async_copy(
            xbuf.at[b], out_hbm.at[pl.ds(rbase + (j - NBUF) * BLK, BLK)],
            ssem.at[b],
        ).wait()


def kernel(x, coords, emb1, emb2):
    c1 = coords[:, 0]
    c2 = coords[:, 1]
    zeros = jnp.zeros((NTAB, HALF), jnp.float32)
    taba = jnp.concatenate([emb1, zeros], axis=1)
    tabb = jnp.concatenate([zeros, emb2], axis=1)
    return _emb_kernel(x, c1, c2, taba, tabb)
